# baseline (speedup 1.0000x reference)
import jax
import jax.numpy as jnp
from jax import lax
from jax.experimental import pallas as pl
from jax.experimental.pallas import tpu as pltpu
from jax.experimental.pallas import tpu_sc as plsc

N = 10000
E = 320000
C = 128
DE = 16
H = 32
PK = 8
EP = E // PK

NC = 2
NS = 16
NW = NC * NS

T = E // NW
K = 64
NFULL = T // K
KT = T - NFULL * K
N_PAD = 10240
ROWS_PER_TILE = N_PAD // NS

BE8 = 800


def _radial_body(ef8_ref, w1b_ref, b1b_ref, w2_ref, b2_ref, out_ref):
    h = jnp.maximum(
        jnp.dot(ef8_ref[...], w1b_ref[...], preferred_element_type=jnp.float32)
        + b1b_ref[...][None, :], 0.0)
    for j in range(PK):
        r = (jnp.dot(h[:, H * j:H * (j + 1)], w2_ref[...],
                     preferred_element_type=jnp.float32)
             + b2_ref[...][None, :])
        out_ref[j] = r


def _radial(ef8, w1b, b1b, w2, b2):
    return pl.pallas_call(
        _radial_body,
        grid=(EP // BE8,),
        in_specs=[
            pl.BlockSpec((BE8, PK * DE), lambda i: (i, 0)),
            pl.BlockSpec((PK * DE, PK * H), lambda i: (0, 0)),
            pl.BlockSpec((PK * H,), lambda i: (0,)),
            pl.BlockSpec((H, C), lambda i: (0, 0)),
            pl.BlockSpec((C,), lambda i: (0,)),
        ],
        out_specs=pl.BlockSpec((PK, BE8, C), lambda i: (0, i, 0)),
        out_shape=jax.ShapeDtypeStruct((PK, EP, C), jnp.float32),
    )(ef8, w1b, b1b, w2, b2)


def _mul_rows(rbv, xg, bas, nrows):
    def mrow(i, _):
        b16 = plsc.load_gather(bas, [jnp.full((16,), i, jnp.int32)])
        for j in range(C // 16):
            sl = pl.ds(j * 16, 16)
            rbv[i, sl] = rbv[i, sl] * xg[i, sl] * b16
        return _
    lax.fori_loop(0, nrows, mrow, None)


def _scatter_body(x_hbm, rb_hbm, src_hbm, dst_hbm, bas_hbm, out_hbm,
                  isrc_all, idst0, idst1, xg0, xg1, rbv0, rbv1, bb0, bb1,
                  idst_t, xg_t, rbv_t, bb_t, agg_sh,
                  sem_g0, sem_g1, sem_r0, sem_r1, sem_i0, sem_i1,
                  sem_b0, sem_b1):
    c = lax.axis_index("c")
    s = lax.axis_index("s")
    wid = c * NS + s
    tbase = wid * T

    idst = (idst0, idst1)
    xg = (xg0, xg1)
    rbv = (rbv0, rbv1)
    bb = (bb0, bb1)
    sem_g = (sem_g0, sem_g1)
    sem_r = (sem_r0, sem_r1)
    sem_i = (sem_i0, sem_i1)
    sem_b = (sem_b0, sem_b1)

    def zrow(i, _):
        for j in range(C // 16):
            xg0[i, pl.ds(j * 16, 16)] = jnp.zeros((16,), jnp.float32)
        return _
    lax.fori_loop(0, K, zrow, None)
    row0 = s * ROWS_PER_TILE
    for z in range(ROWS_PER_TILE // K):
        pltpu.sync_copy(xg0, agg_sh.at[pl.ds(row0 + z * K, K)])
    plsc.subcore_barrier()

    pltpu.sync_copy(src_hbm.at[pl.ds(tbase, T)], isrc_all)

    def start(ci, b):
        base = tbase + ci * K
        pltpu.async_copy(dst_hbm.at[pl.ds(base, K)], idst[b], sem_i[b])
        pltpu.async_copy(bas_hbm.at[pl.ds(base, K)], bb[b], sem_b[b])
        pltpu.async_copy(rb_hbm.at[pl.ds(base, K)], rbv[b], sem_r[b])
        pltpu.async_copy(x_hbm.at[isrc_all.at[pl.ds(ci * K, K)]], xg[b],
                         sem_g[b])

    def finish(b):
        pltpu.make_async_copy(rb_hbm.at[pl.ds(0, K)], rbv[b], sem_r[b]).wait()
        pltpu.make_async_copy(rb_hbm.at[pl.ds(0, K)], xg[b], sem_g[b]).wait()
        pltpu.make_async_copy(bas_hbm.at[pl.ds(0, K)], bb[b], sem_b[b]).wait()
        _mul_rows(rbv[b], xg[b], bb[b], K)
        pltpu.make_async_copy(dst_hbm.at[pl.ds(0, K)], idst[b], sem_i[b]).wait()
        pltpu.sync_copy(rbv[b], agg_sh.at[idst[b]], add=True)

    start(0, 0)
    start(1, 1)

    def pair(i, _):
        c0 = 2 * i
        finish(0)

        @pl.when(c0 + 2 < NFULL)
        def _s0():
            start(c0 + 2, 0)
        finish(1)

        @pl.when(c0 + 3 < NFULL)
        def _s1():
            start(c0 + 3, 1)
        return _
    lax.fori_loop(0, NFULL // 2, pair, None)

    tb = tbase + NFULL * K
    pltpu.sync_copy(dst_hbm.at[pl.ds(tb, KT)], idst_t)
    pltpu.sync_copy(bas_hbm.at[pl.ds(tb, KT)], bb_t)
    pltpu.async_copy(x_hbm.at[isrc_all.at[pl.ds(NFULL * K, KT)]], xg_t,
                     sem_g0).wait()
    pltpu.sync_copy(rb_hbm.at[pl.ds(tb, KT)], rbv_t)
    _mul_rows(rbv_t, xg_t, bb_t, KT)
    pltpu.sync_copy(rbv_t, agg_sh.at[idst_t], add=True)

    plsc.subcore_barrier()
    pltpu.sync_copy(agg_sh.at[pl.ds(row0, ROWS_PER_TILE)],
                    out_hbm.at[c, pl.ds(row0, ROWS_PER_TILE)])


def _sc_scatter(x, rb, src, dst, bas):
    mesh = plsc.VectorSubcoreMesh(core_axis_name="c", subcore_axis_name="s")
    f = pl.kernel(
        _scatter_body,
        out_type=jax.ShapeDtypeStruct((NC, N_PAD, C), jnp.float32),
        mesh=mesh,
        compiler_params=pltpu.CompilerParams(needs_layout_passes=False),
        scratch_types=[
            pltpu.VMEM((T,), jnp.int32),
            pltpu.VMEM((K,), jnp.int32),
            pltpu.VMEM((K,), jnp.int32),
            pltpu.VMEM((K, C), jnp.float32),
            pltpu.VMEM((K, C), jnp.float32),
            pltpu.VMEM((K, C), jnp.float32),
            pltpu.VMEM((K, C), jnp.float32),
            pltpu.VMEM((K,), jnp.float32),
            pltpu.VMEM((K,), jnp.float32),
            pltpu.VMEM((KT,), jnp.int32),
            pltpu.VMEM((KT, C), jnp.float32),
            pltpu.VMEM((KT, C), jnp.float32),
            pltpu.VMEM((KT,), jnp.float32),
            pltpu.VMEM_SHARED((N_PAD, C), jnp.float32),
            pltpu.SemaphoreType.DMA,
            pltpu.SemaphoreType.DMA,
            pltpu.SemaphoreType.DMA,
            pltpu.SemaphoreType.DMA,
            pltpu.SemaphoreType.DMA,
            pltpu.SemaphoreType.DMA,
            pltpu.SemaphoreType.DMA,
            pltpu.SemaphoreType.DMA,
        ],
    )
    return f(x, rb, src, dst, bas)


def _node_body(p_ref, x_ref, dens_ref, wc_ref, ws_ref, wn_ref, bn_ref,
               g_ref, b_ref, wl_ref, out_ref):
    agg = p_ref[0] + p_ref[1]
    u = (jnp.dot(agg, wc_ref[...], preferred_element_type=jnp.float32)
         + jnp.dot(x_ref[...], ws_ref[...], preferred_element_type=jnp.float32))
    norm = jnp.abs(u) + 1e-6
    phase = u / norm
    mu = jnp.mean(norm, axis=-1, keepdims=True)
    var = jnp.mean((norm - mu) ** 2, axis=-1, keepdims=True)
    nln = (norm - mu) * lax.rsqrt(var + 1e-5) * g_ref[...][None, :] \
        + b_ref[...][None, :]
    t = jnp.maximum(
        jnp.dot(nln, wn_ref[...], preferred_element_type=jnp.float32)
        + bn_ref[...][None, :], 0.0)
    upd = jnp.dot(t * phase, wl_ref[...], preferred_element_type=jnp.float32)
    out_ref[...] = dens_ref[...] + upd


def _node_pipeline(partial, x, dens, wc, ws, wn, bn, g, b, wl):
    BN = 1000
    return pl.pallas_call(
        _node_body,
        grid=(N // BN,),
        in_specs=[
            pl.BlockSpec((NC, BN, C), lambda i: (0, i, 0)),
            pl.BlockSpec((BN, C), lambda i: (i, 0)),
            pl.BlockSpec((BN, C), lambda i: (i, 0)),
            pl.BlockSpec((C, C), lambda i: (0, 0)),
            pl.BlockSpec((C, C), lambda i: (0, 0)),
            pl.BlockSpec((C, C), lambda i: (0, 0)),
            pl.BlockSpec((C,), lambda i: (0,)),
            pl.BlockSpec((C,), lambda i: (0,)),
            pl.BlockSpec((C,), lambda i: (0,)),
            pl.BlockSpec((C, C), lambda i: (0, 0)),
        ],
        out_specs=pl.BlockSpec((BN, C), lambda i: (i, 0)),
        out_shape=jax.ShapeDtypeStruct((N, C), jnp.float32),
    )(partial, x, dens, wc, ws, wn, bn, g, b, wl)


def kernel(node_features, density_features, edge_features, edge_index, basis,
           W_rad1, b_rad1, W_rad2, b_rad2, W_conv, W_self,
           W_norm, b_norm, ln_g, ln_b, W_lin):
    ef8 = edge_features.reshape(EP, PK * DE)
    w1b = jnp.kron(jnp.eye(PK, dtype=jnp.float32), W_rad1)
    b1b = jnp.tile(b_rad1, PK)
    src_p = edge_index[0].reshape(EP, PK).T.reshape(E)
    dst_p = edge_index[1].reshape(EP, PK).T.reshape(E)
    bas_p = basis.reshape(EP, PK).T.reshape(E)

    rb = _radial(ef8, w1b, b1b, W_rad2, b_rad2).reshape(E, C)
    partial = _sc_scatter(node_features, rb, src_p, dst_p, bas_p)
    return _node_pipeline(partial, node_features, density_features,
                          W_conv, W_self, W_norm, b_norm, ln_g, ln_b, W_lin)


# speedup vs baseline: 2.4825x; 2.4825x over previous
import jax
import jax.numpy as jnp
from jax import lax
from jax.experimental import pallas as pl
from jax.experimental.pallas import tpu as pltpu
from jax.experimental.pallas import tpu_sc as plsc

N = 10000
E = 320000
C = 128
DE = 16
H = 32

NC = 2
NS = 16
NW = NC * NS

T = E // NW
K = 64
NFULL = T // K
KT = T - NFULL * K
N_PAD = 10240
ROWS_PER_TILE = N_PAD // NS

BEE = 6400

_TDN = (((0,), (0,)), ((), ()))


def _radial_body(eft_ref, bast_ref, w1_ref, b1c_ref, w2_ref, b2r_ref, out_ref):
    ht = jnp.maximum(
        lax.dot_general(w1_ref[...], eft_ref[...], _TDN,
                        preferred_element_type=jnp.float32)
        + b1c_ref[...], 0.0)
    hb = ht * bast_ref[...]
    r = lax.dot_general(hb, w2_ref[...], _TDN,
                        preferred_element_type=jnp.float32)
    r = r + lax.dot_general(bast_ref[...], b2r_ref[...], _TDN,
                            preferred_element_type=jnp.float32)
    out_ref[...] = r


def _radial(eft, bast, w1, b1c, w2, b2r):
    return pl.pallas_call(
        _radial_body,
        grid=(E // BEE,),
        in_specs=[
            pl.BlockSpec((DE, BEE), lambda i: (0, i)),
            pl.BlockSpec((1, BEE), lambda i: (0, i)),
            pl.BlockSpec((DE, H), lambda i: (0, 0)),
            pl.BlockSpec((H, 1), lambda i: (0, 0)),
            pl.BlockSpec((H, C), lambda i: (0, 0)),
            pl.BlockSpec((1, C), lambda i: (0, 0)),
        ],
        out_specs=pl.BlockSpec((BEE, C), lambda i: (i, 0)),
        out_shape=jax.ShapeDtypeStruct((E, C), jnp.float32),
    )(eft, bast, w1, b1c, w2, b2r)


def _mul_rows(rbv, xg, nrows):
    def mrow(i, _):
        for j in range(C // 16):
            sl = pl.ds(j * 16, 16)
            rbv[i, sl] = rbv[i, sl] * xg[i, sl]
        return _
    lax.fori_loop(0, nrows, mrow, None)


def _scatter_body(x_hbm, rb_hbm, src_hbm, dst_hbm, out_hbm,
                  isrc_all, idst0, idst1, xg0, xg1, rbv0, rbv1,
                  idst_t, xg_t, rbv_t, agg_sh,
                  sem_g0, sem_g1, sem_r0, sem_r1, sem_i0, sem_i1):
    c = lax.axis_index("c")
    s = lax.axis_index("s")
    wid = c * NS + s
    tbase = wid * T

    idst = (idst0, idst1)
    xg = (xg0, xg1)
    rbv = (rbv0, rbv1)
    sem_g = (sem_g0, sem_g1)
    sem_r = (sem_r0, sem_r1)
    sem_i = (sem_i0, sem_i1)

    def zrow(i, _):
        for j in range(C // 16):
            xg0[i, pl.ds(j * 16, 16)] = jnp.zeros((16,), jnp.float32)
        return _
    lax.fori_loop(0, K, zrow, None)
    row0 = s * ROWS_PER_TILE
    for z in range(ROWS_PER_TILE // K):
        pltpu.sync_copy(xg0, agg_sh.at[pl.ds(row0 + z * K, K)])
    plsc.subcore_barrier()

    pltpu.sync_copy(src_hbm.at[pl.ds(tbase, T)], isrc_all)

    def start(ci, b):
        base = tbase + ci * K
        pltpu.async_copy(dst_hbm.at[pl.ds(base, K)], idst[b], sem_i[b])
        pltpu.async_copy(rb_hbm.at[pl.ds(base, K)], rbv[b], sem_r[b])
        pltpu.async_copy(x_hbm.at[isrc_all.at[pl.ds(ci * K, K)]], xg[b],
                         sem_g[b])

    def finish(b):
        pltpu.make_async_copy(rb_hbm.at[pl.ds(0, K)], rbv[b], sem_r[b]).wait()
        pltpu.make_async_copy(rb_hbm.at[pl.ds(0, K)], xg[b], sem_g[b]).wait()
        _mul_rows(rbv[b], xg[b], K)
        pltpu.make_async_copy(dst_hbm.at[pl.ds(0, K)], idst[b], sem_i[b]).wait()
        pltpu.sync_copy(rbv[b], agg_sh.at[idst[b]], add=True)

    start(0, 0)
    start(1, 1)

    def pair(i, _):
        c0 = 2 * i
        finish(0)

        @pl.when(c0 + 2 < NFULL)
        def _s0():
            start(c0 + 2, 0)
        finish(1)

        @pl.when(c0 + 3 < NFULL)
        def _s1():
            start(c0 + 3, 1)
        return _
    lax.fori_loop(0, NFULL // 2, pair, None)

    tb = tbase + NFULL * K
    pltpu.sync_copy(dst_hbm.at[pl.ds(tb, KT)], idst_t)
    pltpu.async_copy(x_hbm.at[isrc_all.at[pl.ds(NFULL * K, KT)]], xg_t,
                     sem_g0).wait()
    pltpu.sync_copy(rb_hbm.at[pl.ds(tb, KT)], rbv_t)
    _mul_rows(rbv_t, xg_t, KT)
    pltpu.sync_copy(rbv_t, agg_sh.at[idst_t], add=True)

    plsc.subcore_barrier()
    pltpu.sync_copy(agg_sh.at[pl.ds(row0, ROWS_PER_TILE)],
                    out_hbm.at[c, pl.ds(row0, ROWS_PER_TILE)])


def _sc_scatter(x, rb, src, dst):
    mesh = plsc.VectorSubcoreMesh(core_axis_name="c", subcore_axis_name="s")
    f = pl.kernel(
        _scatter_body,
        out_type=jax.ShapeDtypeStruct((NC, N_PAD, C), jnp.float32),
        mesh=mesh,
        scratch_types=[
            pltpu.VMEM((T,), jnp.int32),
            pltpu.VMEM((K,), jnp.int32),
            pltpu.VMEM((K,), jnp.int32),
            pltpu.VMEM((K, C), jnp.float32),
            pltpu.VMEM((K, C), jnp.float32),
            pltpu.VMEM((K, C), jnp.float32),
            pltpu.VMEM((K, C), jnp.float32),
            pltpu.VMEM((KT,), jnp.int32),
            pltpu.VMEM((KT, C), jnp.float32),
            pltpu.VMEM((KT, C), jnp.float32),
            pltpu.VMEM_SHARED((N_PAD, C), jnp.float32),
            pltpu.SemaphoreType.DMA,
            pltpu.SemaphoreType.DMA,
            pltpu.SemaphoreType.DMA,
            pltpu.SemaphoreType.DMA,
            pltpu.SemaphoreType.DMA,
            pltpu.SemaphoreType.DMA,
        ],
    )
    return f(x, rb, src, dst)


def _node_body(p_ref, x_ref, dens_ref, wc_ref, ws_ref, wn_ref, bn_ref,
               g_ref, b_ref, wl_ref, out_ref):
    agg = p_ref[0] + p_ref[1]
    u = (jnp.dot(agg, wc_ref[...], preferred_element_type=jnp.float32)
         + jnp.dot(x_ref[...], ws_ref[...], preferred_element_type=jnp.float32))
    norm = jnp.abs(u) + 1e-6
    phase = u / norm
    mu = jnp.mean(norm, axis=-1, keepdims=True)
    var = jnp.mean((norm - mu) ** 2, axis=-1, keepdims=True)
    nln = (norm - mu) * lax.rsqrt(var + 1e-5) * g_ref[...][None, :] \
        + b_ref[...][None, :]
    t = jnp.maximum(
        jnp.dot(nln, wn_ref[...], preferred_element_type=jnp.float32)
        + bn_ref[...][None, :], 0.0)
    upd = jnp.dot(t * phase, wl_ref[...], preferred_element_type=jnp.float32)
    out_ref[...] = dens_ref[...] + upd


def _node_pipeline(partial, x, dens, wc, ws, wn, bn, g, b, wl):
    BN = 1000
    return pl.pallas_call(
        _node_body,
        grid=(N // BN,),
        in_specs=[
            pl.BlockSpec((NC, BN, C), lambda i: (0, i, 0)),
            pl.BlockSpec((BN, C), lambda i: (i, 0)),
            pl.BlockSpec((BN, C), lambda i: (i, 0)),
            pl.BlockSpec((C, C), lambda i: (0, 0)),
            pl.BlockSpec((C, C), lambda i: (0, 0)),
            pl.BlockSpec((C, C), lambda i: (0, 0)),
            pl.BlockSpec((C,), lambda i: (0,)),
            pl.BlockSpec((C,), lambda i: (0,)),
            pl.BlockSpec((C,), lambda i: (0,)),
            pl.BlockSpec((C, C), lambda i: (0, 0)),
        ],
        out_specs=pl.BlockSpec((BN, C), lambda i: (i, 0)),
        out_shape=jax.ShapeDtypeStruct((N, C), jnp.float32),
    )(partial, x, dens, wc, ws, wn, bn, g, b, wl)


def kernel(node_features, density_features, edge_features, edge_index, basis,
           W_rad1, b_rad1, W_rad2, b_rad2, W_conv, W_self,
           W_norm, b_norm, ln_g, ln_b, W_lin):
    eft = edge_features.T
    bast = basis.T
    b1c = b_rad1.reshape(H, 1)
    b2r = b_rad2.reshape(1, C)
    src = edge_index[0]
    dst = edge_index[1]

    rb = _radial(eft, bast, W_rad1, b1c, W_rad2, b2r)
    partial = _sc_scatter(node_features, rb, src, dst)
    return _node_pipeline(partial, node_features, density_features,
                          W_conv, W_self, W_norm, b_norm, ln_g, ln_b, W_lin)


# speedup vs baseline: 2.5579x; 1.0304x over previous
import jax
import jax.numpy as jnp
from jax import lax
from jax.experimental import pallas as pl
from jax.experimental.pallas import tpu as pltpu
from jax.experimental.pallas import tpu_sc as plsc

N = 10000
E = 320000
C = 128
DE = 16
H = 32

NC = 2
NS = 16
NW = NC * NS

T = E // NW
K = 64
NFULL = T // K
KT = T - NFULL * K
N_PAD = 10240
ROWS_PER_TILE = N_PAD // NS

BEE = 6400

_TDN = (((0,), (0,)), ((), ()))


def _radial_body(eft_ref, bast_ref, w1_ref, b1c_ref, w2_ref, b2r_ref, out_ref):
    ht = jnp.maximum(
        lax.dot_general(w1_ref[...], eft_ref[...], _TDN,
                        preferred_element_type=jnp.float32)
        + b1c_ref[...], 0.0)
    hb = ht * bast_ref[...]
    r = lax.dot_general(hb, w2_ref[...], _TDN,
                        preferred_element_type=jnp.float32)
    r = r + lax.dot_general(bast_ref[...], b2r_ref[...], _TDN,
                            preferred_element_type=jnp.float32)
    out_ref[...] = r


def _radial(eft, bast, w1, b1c, w2, b2r):
    return pl.pallas_call(
        _radial_body,
        grid=(E // BEE,),
        in_specs=[
            pl.BlockSpec((DE, BEE), lambda i: (0, i)),
            pl.BlockSpec((1, BEE), lambda i: (0, i)),
            pl.BlockSpec((DE, H), lambda i: (0, 0)),
            pl.BlockSpec((H, 1), lambda i: (0, 0)),
            pl.BlockSpec((H, C), lambda i: (0, 0)),
            pl.BlockSpec((1, C), lambda i: (0, 0)),
        ],
        out_specs=pl.BlockSpec((BEE, C), lambda i: (i, 0)),
        out_shape=jax.ShapeDtypeStruct((E, C), jnp.float32),
    )(eft, bast, w1, b1c, w2, b2r)


def _mul_rows(rbv, xg, nrows):
    def mrow(i, _):
        for j in range(C // 16):
            sl = pl.ds(j * 16, 16)
            rbv[i, sl] = rbv[i, sl] * xg[i, sl]
        return _
    lax.fori_loop(0, nrows, mrow, None)


def _scatter_body(x_hbm, rb_hbm, ei_hbm, out_hbm,
                  isrc_all, idst0, idst1, xg0, xg1, rbv0, rbv1,
                  idst_t, xg_t, rbv_t, agg_sh,
                  sem_g0, sem_g1, sem_r0, sem_r1, sem_i0, sem_i1):
    c = lax.axis_index("c")
    s = lax.axis_index("s")
    wid = c * NS + s
    tbase = wid * T

    idst = (idst0, idst1)
    xg = (xg0, xg1)
    rbv = (rbv0, rbv1)
    sem_g = (sem_g0, sem_g1)
    sem_r = (sem_r0, sem_r1)
    sem_i = (sem_i0, sem_i1)

    def zrow(i, _):
        for j in range(C // 16):
            xg0[i, pl.ds(j * 16, 16)] = jnp.zeros((16,), jnp.float32)
        return _
    lax.fori_loop(0, K, zrow, None)
    row0 = s * ROWS_PER_TILE
    for z in range(ROWS_PER_TILE // K):
        pltpu.sync_copy(xg0, agg_sh.at[pl.ds(row0 + z * K, K)])
    plsc.subcore_barrier()

    pltpu.sync_copy(ei_hbm.at[pl.ds(tbase, T)], isrc_all)

    def start(ci, b):
        base = tbase + ci * K
        pltpu.async_copy(ei_hbm.at[pl.ds(E + base, K)], idst[b], sem_i[b])
        pltpu.async_copy(rb_hbm.at[pl.ds(base, K)], rbv[b], sem_r[b])
        pltpu.async_copy(x_hbm.at[isrc_all.at[pl.ds(ci * K, K)]], xg[b],
                         sem_g[b])

    def finish(b):
        pltpu.make_async_copy(rb_hbm.at[pl.ds(0, K)], rbv[b], sem_r[b]).wait()
        pltpu.make_async_copy(x_hbm.at[pl.ds(0, K)], xg[b], sem_g[b]).wait()
        _mul_rows(rbv[b], xg[b], K)
        pltpu.make_async_copy(ei_hbm.at[pl.ds(0, K)], idst[b], sem_i[b]).wait()
        pltpu.sync_copy(rbv[b], agg_sh.at[idst[b]], add=True)

    start(0, 0)
    start(1, 1)

    def pair(i, _):
        c0 = 2 * i
        finish(0)

        @pl.when(c0 + 2 < NFULL)
        def _s0():
            start(c0 + 2, 0)
        finish(1)

        @pl.when(c0 + 3 < NFULL)
        def _s1():
            start(c0 + 3, 1)
        return _
    lax.fori_loop(0, NFULL // 2, pair, None)

    tb = tbase + NFULL * K
    pltpu.sync_copy(ei_hbm.at[pl.ds(E + tb, KT)], idst_t)
    pltpu.async_copy(x_hbm.at[isrc_all.at[pl.ds(NFULL * K, KT)]], xg_t,
                     sem_g0).wait()
    pltpu.sync_copy(rb_hbm.at[pl.ds(tb, KT)], rbv_t)
    _mul_rows(rbv_t, xg_t, KT)
    pltpu.sync_copy(rbv_t, agg_sh.at[idst_t], add=True)

    plsc.subcore_barrier()
    pltpu.sync_copy(agg_sh.at[pl.ds(row0, ROWS_PER_TILE)],
                    out_hbm.at[c, pl.ds(row0, ROWS_PER_TILE)])


def _sc_scatter(x, rb, ei):
    mesh = plsc.VectorSubcoreMesh(core_axis_name="c", subcore_axis_name="s")
    f = pl.kernel(
        _scatter_body,
        out_type=jax.ShapeDtypeStruct((NC, N_PAD, C), jnp.float32),
        mesh=mesh,
        scratch_types=[
            pltpu.VMEM((T,), jnp.int32),
            pltpu.VMEM((K,), jnp.int32),
            pltpu.VMEM((K,), jnp.int32),
            pltpu.VMEM((K, C), jnp.float32),
            pltpu.VMEM((K, C), jnp.float32),
            pltpu.VMEM((K, C), jnp.float32),
            pltpu.VMEM((K, C), jnp.float32),
            pltpu.VMEM((KT,), jnp.int32),
            pltpu.VMEM((KT, C), jnp.float32),
            pltpu.VMEM((KT, C), jnp.float32),
            pltpu.VMEM_SHARED((N_PAD, C), jnp.float32),
            pltpu.SemaphoreType.DMA,
            pltpu.SemaphoreType.DMA,
            pltpu.SemaphoreType.DMA,
            pltpu.SemaphoreType.DMA,
            pltpu.SemaphoreType.DMA,
            pltpu.SemaphoreType.DMA,
        ],
    )
    return f(x, rb, ei)


def _node_body(p_ref, x_ref, dens_ref, wc_ref, ws_ref, wn_ref, bn_ref,
               g_ref, b_ref, wl_ref, out_ref):
    agg = p_ref[0] + p_ref[1]
    u = (jnp.dot(agg, wc_ref[...], preferred_element_type=jnp.float32)
         + jnp.dot(x_ref[...], ws_ref[...], preferred_element_type=jnp.float32))
    norm = jnp.abs(u) + 1e-6
    phase = u / norm
    mu = jnp.mean(norm, axis=-1, keepdims=True)
    var = jnp.mean((norm - mu) ** 2, axis=-1, keepdims=True)
    nln = (norm - mu) * lax.rsqrt(var + 1e-5) * g_ref[...][None, :] \
        + b_ref[...][None, :]
    t = jnp.maximum(
        jnp.dot(nln, wn_ref[...], preferred_element_type=jnp.float32)
        + bn_ref[...][None, :], 0.0)
    upd = jnp.dot(t * phase, wl_ref[...], preferred_element_type=jnp.float32)
    out_ref[...] = dens_ref[...] + upd


def _node_pipeline(partial, x, dens, wc, ws, wn, bn, g, b, wl):
    BN = 1000
    return pl.pallas_call(
        _node_body,
        grid=(N // BN,),
        in_specs=[
            pl.BlockSpec((NC, BN, C), lambda i: (0, i, 0)),
            pl.BlockSpec((BN, C), lambda i: (i, 0)),
            pl.BlockSpec((BN, C), lambda i: (i, 0)),
            pl.BlockSpec((C, C), lambda i: (0, 0)),
            pl.BlockSpec((C, C), lambda i: (0, 0)),
            pl.BlockSpec((C, C), lambda i: (0, 0)),
            pl.BlockSpec((C,), lambda i: (0,)),
            pl.BlockSpec((C,), lambda i: (0,)),
            pl.BlockSpec((C,), lambda i: (0,)),
            pl.BlockSpec((C, C), lambda i: (0, 0)),
        ],
        out_specs=pl.BlockSpec((BN, C), lambda i: (i, 0)),
        out_shape=jax.ShapeDtypeStruct((N, C), jnp.float32),
    )(partial, x, dens, wc, ws, wn, bn, g, b, wl)


def kernel(node_features, density_features, edge_features, edge_index, basis,
           W_rad1, b_rad1, W_rad2, b_rad2, W_conv, W_self,
           W_norm, b_norm, ln_g, ln_b, W_lin):
    eft = edge_features.T
    bast = basis.T
    b1c = b_rad1.reshape(H, 1)
    b2r = b_rad2.reshape(1, C)
    ei = edge_index.reshape(2 * E)

    rb = _radial(eft, bast, W_rad1, b1c, W_rad2, b2r)
    partial = _sc_scatter(node_features, rb, ei)
    return _node_pipeline(partial, node_features, density_features,
                          W_conv, W_self, W_norm, b_norm, ln_g, ln_b, W_lin)


# speedup vs baseline: 2.6780x; 1.0470x over previous
import jax
import jax.numpy as jnp
from jax import lax
from jax.experimental import pallas as pl
from jax.experimental.pallas import tpu as pltpu
from jax.experimental.pallas import tpu_sc as plsc

N = 10000
E = 320000
C = 128
DE = 16
H = 32

NC = 2
NS = 16
NW = NC * NS

NSLAB = 2
EH = E // NSLAB
T = EH // NW
K = 64
NFULL = T // K
KT = T - NFULL * K
N_PAD = 10240
ROWS_PER_TILE = N_PAD // NS

BEE = 6400

_TDN = (((0,), (0,)), ((), ()))


def _radial_body(eft_ref, bast_ref, w1_ref, b1c_ref, w2_ref, b2r_ref, out_ref):
    ht = jnp.maximum(
        lax.dot_general(w1_ref[...], eft_ref[...], _TDN,
                        preferred_element_type=jnp.float32)
        + b1c_ref[...], 0.0)
    hb = ht * bast_ref[...]
    r = lax.dot_general(hb, w2_ref[...], _TDN,
                        preferred_element_type=jnp.float32)
    r = r + lax.dot_general(bast_ref[...], b2r_ref[...], _TDN,
                            preferred_element_type=jnp.float32)
    out_ref[...] = r


def _radial(eft, bast, w1, b1c, w2, b2r, off):
    return pl.pallas_call(
        _radial_body,
        grid=(EH // BEE,),
        in_specs=[
            pl.BlockSpec((DE, BEE), lambda i: (0, i + off)),
            pl.BlockSpec((1, BEE), lambda i: (0, i + off)),
            pl.BlockSpec((DE, H), lambda i: (0, 0)),
            pl.BlockSpec((H, 1), lambda i: (0, 0)),
            pl.BlockSpec((H, C), lambda i: (0, 0)),
            pl.BlockSpec((1, C), lambda i: (0, 0)),
        ],
        out_specs=pl.BlockSpec((BEE, C), lambda i: (i, 0)),
        out_shape=jax.ShapeDtypeStruct((EH, C), jnp.float32),
    )(eft, bast, w1, b1c, w2, b2r)


def _mul_rows(rbv, xg, nrows):
    def mrow(i, _):
        for j in range(C // 16):
            sl = pl.ds(j * 16, 16)
            rbv[i, sl] = rbv[i, sl] * xg[i, sl]
        return _
    lax.fori_loop(0, nrows, mrow, None)


def _make_scatter_body(e0):
  def _scatter_body(x_hbm, rb_hbm, ei_hbm, out_hbm,
                    isrc_all, idst0, idst1, xg0, xg1, rbv0, rbv1,
                    idst_t, xg_t, rbv_t, agg_sh,
                    sem_g0, sem_g1, sem_r0, sem_r1, sem_i0, sem_i1):
    c = lax.axis_index("c")
    s = lax.axis_index("s")
    wid = c * NS + s
    tbase = wid * T
    gbase = e0 + tbase

    idst = (idst0, idst1)
    xg = (xg0, xg1)
    rbv = (rbv0, rbv1)
    sem_g = (sem_g0, sem_g1)
    sem_r = (sem_r0, sem_r1)
    sem_i = (sem_i0, sem_i1)

    def zrow(i, _):
        for j in range(C // 16):
            xg0[i, pl.ds(j * 16, 16)] = jnp.zeros((16,), jnp.float32)
        return _
    lax.fori_loop(0, K, zrow, None)
    row0 = s * ROWS_PER_TILE
    for z in range(ROWS_PER_TILE // K):
        pltpu.sync_copy(xg0, agg_sh.at[pl.ds(row0 + z * K, K)])
    plsc.subcore_barrier()

    pltpu.sync_copy(ei_hbm.at[pl.ds(gbase, T)], isrc_all)

    def start(ci, b):
        base = tbase + ci * K
        pltpu.async_copy(ei_hbm.at[pl.ds(E + e0 + base, K)], idst[b], sem_i[b])
        pltpu.async_copy(rb_hbm.at[pl.ds(base, K)], rbv[b], sem_r[b])
        pltpu.async_copy(x_hbm.at[isrc_all.at[pl.ds(ci * K, K)]], xg[b],
                         sem_g[b])

    def finish(b):
        pltpu.make_async_copy(rb_hbm.at[pl.ds(0, K)], rbv[b], sem_r[b]).wait()
        pltpu.make_async_copy(x_hbm.at[pl.ds(0, K)], xg[b], sem_g[b]).wait()
        _mul_rows(rbv[b], xg[b], K)
        pltpu.make_async_copy(ei_hbm.at[pl.ds(0, K)], idst[b], sem_i[b]).wait()
        pltpu.sync_copy(rbv[b], agg_sh.at[idst[b]], add=True)

    start(0, 0)
    start(1, 1)

    def pair(i, _):
        c0 = 2 * i
        finish(0)

        @pl.when(c0 + 2 < NFULL)
        def _s0():
            start(c0 + 2, 0)
        finish(1)

        @pl.when(c0 + 3 < NFULL)
        def _s1():
            start(c0 + 3, 1)
        return _
    lax.fori_loop(0, NFULL // 2, pair, None)

    tb = tbase + NFULL * K
    pltpu.sync_copy(ei_hbm.at[pl.ds(E + e0 + tb, KT)], idst_t)
    pltpu.async_copy(x_hbm.at[isrc_all.at[pl.ds(NFULL * K, KT)]], xg_t,
                     sem_g0).wait()
    pltpu.sync_copy(rb_hbm.at[pl.ds(tb, KT)], rbv_t)
    _mul_rows(rbv_t, xg_t, KT)
    pltpu.sync_copy(rbv_t, agg_sh.at[idst_t], add=True)

    plsc.subcore_barrier()
    pltpu.sync_copy(agg_sh.at[pl.ds(row0, ROWS_PER_TILE)],
                    out_hbm.at[c, pl.ds(row0, ROWS_PER_TILE)])

  return _scatter_body


def _sc_scatter(x, rb, ei, e0):
    mesh = plsc.VectorSubcoreMesh(core_axis_name="c", subcore_axis_name="s")
    f = pl.kernel(
        _make_scatter_body(e0),
        out_type=jax.ShapeDtypeStruct((NC, N_PAD, C), jnp.float32),
        mesh=mesh,
        scratch_types=[
            pltpu.VMEM((T,), jnp.int32),
            pltpu.VMEM((K,), jnp.int32),
            pltpu.VMEM((K,), jnp.int32),
            pltpu.VMEM((K, C), jnp.float32),
            pltpu.VMEM((K, C), jnp.float32),
            pltpu.VMEM((K, C), jnp.float32),
            pltpu.VMEM((K, C), jnp.float32),
            pltpu.VMEM((KT,), jnp.int32),
            pltpu.VMEM((KT, C), jnp.float32),
            pltpu.VMEM((KT, C), jnp.float32),
            pltpu.VMEM_SHARED((N_PAD, C), jnp.float32),
            pltpu.SemaphoreType.DMA,
            pltpu.SemaphoreType.DMA,
            pltpu.SemaphoreType.DMA,
            pltpu.SemaphoreType.DMA,
            pltpu.SemaphoreType.DMA,
            pltpu.SemaphoreType.DMA,
        ],
    )
    return f(x, rb, ei)


def _node_body(pa_ref, pb_ref, x_ref, dens_ref, wc_ref, ws_ref, wn_ref, bn_ref,
               g_ref, b_ref, wl_ref, out_ref):
    agg = pa_ref[0] + pa_ref[1] + pb_ref[0] + pb_ref[1]
    u = (jnp.dot(agg, wc_ref[...], preferred_element_type=jnp.float32)
         + jnp.dot(x_ref[...], ws_ref[...], preferred_element_type=jnp.float32))
    norm = jnp.abs(u) + 1e-6
    phase = u / norm
    mu = jnp.mean(norm, axis=-1, keepdims=True)
    var = jnp.mean((norm - mu) ** 2, axis=-1, keepdims=True)
    nln = (norm - mu) * lax.rsqrt(var + 1e-5) * g_ref[...][None, :] \
        + b_ref[...][None, :]
    t = jnp.maximum(
        jnp.dot(nln, wn_ref[...], preferred_element_type=jnp.float32)
        + bn_ref[...][None, :], 0.0)
    upd = jnp.dot(t * phase, wl_ref[...], preferred_element_type=jnp.float32)
    out_ref[...] = dens_ref[...] + upd


def _node_pipeline(pa, pb, x, dens, wc, ws, wn, bn, g, b, wl):
    BN = 1000
    return pl.pallas_call(
        _node_body,
        grid=(N // BN,),
        in_specs=[
            pl.BlockSpec((NC, BN, C), lambda i: (0, i, 0)),
            pl.BlockSpec((NC, BN, C), lambda i: (0, i, 0)),
            pl.BlockSpec((BN, C), lambda i: (i, 0)),
            pl.BlockSpec((BN, C), lambda i: (i, 0)),
            pl.BlockSpec((C, C), lambda i: (0, 0)),
            pl.BlockSpec((C, C), lambda i: (0, 0)),
            pl.BlockSpec((C, C), lambda i: (0, 0)),
            pl.BlockSpec((C,), lambda i: (0,)),
            pl.BlockSpec((C,), lambda i: (0,)),
            pl.BlockSpec((C,), lambda i: (0,)),
            pl.BlockSpec((C, C), lambda i: (0, 0)),
        ],
        out_specs=pl.BlockSpec((BN, C), lambda i: (i, 0)),
        out_shape=jax.ShapeDtypeStruct((N, C), jnp.float32),
    )(pa, pb, x, dens, wc, ws, wn, bn, g, b, wl)


def kernel(node_features, density_features, edge_features, edge_index, basis,
           W_rad1, b_rad1, W_rad2, b_rad2, W_conv, W_self,
           W_norm, b_norm, ln_g, ln_b, W_lin):
    eft = edge_features.T
    bast = basis.T
    b1c = b_rad1.reshape(H, 1)
    b2r = b_rad2.reshape(1, C)
    ei = edge_index.reshape(2 * E)

    rba = _radial(eft, bast, W_rad1, b1c, W_rad2, b2r, 0)
    pa = _sc_scatter(node_features, rba, ei, 0)
    rbb = _radial(eft, bast, W_rad1, b1c, W_rad2, b2r, EH // BEE)
    pb = _sc_scatter(node_features, rbb, ei, EH)
    return _node_pipeline(pa, pb, node_features, density_features,
                          W_conv, W_self, W_norm, b_norm, ln_g, ln_b, W_lin)
